# Initial kernel scaffold; baseline (speedup 1.0000x reference)
import jax
import jax.numpy as jnp
from jax.experimental import pallas as pl


def kernel(feat_0, feat_1, feat_2, feat_3, feat_4, feat_5, feat_6, feat_7, feat_8, feat_9, feat_10, feat_11, feat_12, feat_13, feat_14, feat_15, feat_16, feat_17, feat_18, feat_19, feat_20, feat_21, feat_22, feat_23, tag_0, tag_1, emb_table, wide_weight):
    raise NotImplementedError("write your pallas kernel here")


# speedup vs baseline: 1.3921x; 1.3921x over previous
import functools

import jax
import jax.numpy as jnp
from jax import lax
from jax.experimental import pallas as pl
from jax.experimental.pallas import tpu as pltpu
from jax.experimental.pallas import tpu_sc as plsc

NUM_ID = 24
NUM_TAG = 2
HIST = 20
B = 4096
EMB = 32
NC = 2
NS = 16
NW = NC * NS
BPW = B // NW
NB = 32
NCHUNK = BPW // NB
IDN = NB * NUM_ID
TAGN = NB * NUM_TAG * HIST


def _sc_body(id_idx_hbm, tag_idx_hbm, tagt_idx_hbm, emb_hbm, wide_hbm,
             emb_id_out, emb_tag_out, wide_id_out, wide_tag_out,
             ididx_v, tagidx_v, tagtidx_v, idrows_v, tagrows_v, tagsum_v,
             wide_id_v, wide_tag_v, widesum_v,
             sem1, sem2, sem3, sem4):
  wid = lax.axis_index("s") * NC + lax.axis_index("c")

  for chunk in range(NCHUNK):
    b0 = wid * BPW + chunk * NB
    pltpu.sync_copy(id_idx_hbm.at[pl.ds(b0 * NUM_ID, IDN)], ididx_v)
    pltpu.sync_copy(tag_idx_hbm.at[pl.ds(b0 * NUM_TAG * HIST, TAGN)],
                    tagidx_v)
    pltpu.sync_copy(tagt_idx_hbm.at[pl.ds(b0 * NUM_TAG * HIST, TAGN)],
                    tagtidx_v)
    cp1 = pltpu.async_copy(emb_hbm.at[ididx_v], idrows_v, sem1)
    cp2 = pltpu.async_copy(emb_hbm.at[tagidx_v], tagrows_v, sem2)
    cp3 = pltpu.async_copy(wide_hbm.at[ididx_v], wide_id_v, sem3)
    cp4 = pltpu.async_copy(wide_hbm.at[tagtidx_v], wide_tag_v, sem4)

    cp1.wait()
    pltpu.sync_copy(idrows_v, emb_id_out.at[pl.ds(b0 * NUM_ID, IDN)])
    cp3.wait()
    pltpu.sync_copy(wide_id_v, wide_id_out.at[pl.ds(b0 * NUM_ID, IDN)])

    cp2.wait()

    def ksum(k, carry):
      for t in range(NUM_TAG):
        base = k * (NUM_TAG * HIST) + t * HIST
        acc0 = jnp.zeros((16,), jnp.float32)
        acc1 = jnp.zeros((16,), jnp.float32)
        for j in range(HIST):
          acc0 = acc0 + tagrows_v[base + j, pl.ds(0, 16)]
          acc1 = acc1 + tagrows_v[base + j, pl.ds(16, 16)]
        tagsum_v[k * NUM_TAG + t, pl.ds(0, 16)] = acc0
        tagsum_v[k * NUM_TAG + t, pl.ds(16, 16)] = acc1
      return carry

    lax.fori_loop(0, NB, ksum, 0, unroll=False)
    pltpu.sync_copy(tagsum_v, emb_tag_out.at[pl.ds(b0 * NUM_TAG,
                                                   NB * NUM_TAG)])

    cp4.wait()
    for t in range(NUM_TAG):
      for half in range(NB // 16):
        acc = jnp.zeros((16,), jnp.float32)
        for j in range(HIST):
          acc = acc + wide_tag_v[pl.ds(t * HIST * NB + j * NB + half * 16,
                                       16)]
        widesum_v[pl.ds(t * NB + half * 16, 16)] = acc
      pltpu.sync_copy(widesum_v.at[pl.ds(t * NB, NB)],
                      wide_tag_out.at[pl.ds(t * B + b0, NB)])


@jax.jit
def _run(id_idx2, tag_idx2, tagt_idx2, emb_table, wide_weight):
  mesh = plsc.VectorSubcoreMesh(core_axis_name="c", subcore_axis_name="s",
                                num_cores=NC, num_subcores=NS)
  out_type = [
      jax.ShapeDtypeStruct((B * NUM_ID, EMB), jnp.float32),
      jax.ShapeDtypeStruct((B * NUM_TAG, EMB), jnp.float32),
      jax.ShapeDtypeStruct((B * NUM_ID,), jnp.float32),
      jax.ShapeDtypeStruct((NUM_TAG * B,), jnp.float32),
  ]
  scratch_types = [
      pltpu.VMEM((IDN,), jnp.int32),
      pltpu.VMEM((TAGN,), jnp.int32),
      pltpu.VMEM((TAGN,), jnp.int32),
      pltpu.VMEM((IDN, EMB), jnp.float32),
      pltpu.VMEM((TAGN, EMB), jnp.float32),
      pltpu.VMEM((NB * NUM_TAG, EMB), jnp.float32),
      pltpu.VMEM((IDN,), jnp.float32),
      pltpu.VMEM((TAGN,), jnp.float32),
      pltpu.VMEM((NB * NUM_TAG,), jnp.float32),
      pltpu.SemaphoreType.DMA,
      pltpu.SemaphoreType.DMA,
      pltpu.SemaphoreType.DMA,
      pltpu.SemaphoreType.DMA,
  ]
  run = pl.kernel(_sc_body, out_type=out_type, mesh=mesh,
                  scratch_types=scratch_types,
                  compiler_params=pltpu.CompilerParams(
                      use_tc_tiling_on_sc=False))
  return run(id_idx2, tag_idx2, tagt_idx2, emb_table, wide_weight)


def kernel(feat_0, feat_1, feat_2, feat_3, feat_4, feat_5, feat_6, feat_7,
           feat_8, feat_9, feat_10, feat_11, feat_12, feat_13, feat_14,
           feat_15, feat_16, feat_17, feat_18, feat_19, feat_20, feat_21,
           feat_22, feat_23, tag_0, tag_1, emb_table, wide_weight):
  feats = [feat_0, feat_1, feat_2, feat_3, feat_4, feat_5, feat_6, feat_7,
           feat_8, feat_9, feat_10, feat_11, feat_12, feat_13, feat_14,
           feat_15, feat_16, feat_17, feat_18, feat_19, feat_20, feat_21,
           feat_22, feat_23]
  id_idx = jnp.stack(feats, axis=1).reshape(B * NUM_ID)
  tags = jnp.stack([tag_0, tag_1], axis=1)
  tag_idx = tags.reshape(B * NUM_TAG * HIST)
  tagt_idx = tags.reshape(B // NB, NB, NUM_TAG, HIST).transpose(
      0, 2, 3, 1).reshape(B * NUM_TAG * HIST)

  emb_id, emb_tag, wide_id, wide_tag = _run(id_idx, tag_idx, tagt_idx,
                                            emb_table, wide_weight)
  emb = jnp.concatenate([emb_id.reshape(B, NUM_ID, EMB),
                         emb_tag.reshape(B, NUM_TAG, EMB)], axis=1)
  wide = jnp.concatenate([wide_id.reshape(B, NUM_ID),
                          wide_tag.reshape(NUM_TAG, B).T], axis=1)
  return (wide, emb)


# speedup vs baseline: 1.4149x; 1.0164x over previous
import jax
import jax.numpy as jnp
from jax import lax
from jax.experimental import pallas as pl
from jax.experimental.pallas import tpu as pltpu
from jax.experimental.pallas import tpu_sc as plsc

NUM_ID = 24
NUM_TAG = 2
HIST = 20
TPB = NUM_TAG * HIST
B = 4096
EMB = 32
NC = 2
NS = 16
NW = NC * NS
BPW = B // NW
NB = 16
NCHUNK = BPW // NB
IDN = NB * NUM_ID
TAGN = NB * TPB


def _sc_body(id_idx_hbm, tag_idx_hbm, tagt_idx_hbm, emb_hbm, wide_hbm,
             emb_id_out, emb_tag_out, wide_id_out, wide_tag_out,
             ididx_v, tagidx_v, tagtidx_v, idrows_v, tagrows_v,
             wide_id_v, wide_tag_v, tagsum_v, widesum_v, *sems):
  gs = sems[:8]
  os_ = sems[8:16]
  wid = lax.axis_index("s") * NC + lax.axis_index("c")
  w0 = wid * BPW

  pltpu.sync_copy(id_idx_hbm.at[pl.ds(w0 * NUM_ID, BPW * NUM_ID)], ididx_v)
  pltpu.sync_copy(tag_idx_hbm.at[pl.ds(w0 * TPB, BPW * TPB)], tagidx_v)
  pltpu.sync_copy(tagt_idx_hbm.at[pl.ds(w0 * TPB, BPW * TPB)], tagtidx_v)

  def issue_gathers(c, s):
    i0 = c * IDN
    t0 = c * TAGN
    return [
        pltpu.async_copy(emb_hbm.at[ididx_v.at[pl.ds(i0, IDN)]],
                         idrows_v.at[s], gs[s * 4 + 0]),
        pltpu.async_copy(emb_hbm.at[tagidx_v.at[pl.ds(t0, TAGN)]],
                         tagrows_v.at[s], gs[s * 4 + 1]),
        pltpu.async_copy(wide_hbm.at[ididx_v.at[pl.ds(i0, IDN)]],
                         wide_id_v.at[s], gs[s * 4 + 2]),
        pltpu.async_copy(wide_hbm.at[tagtidx_v.at[pl.ds(t0, TAGN)]],
                         wide_tag_v.at[s], gs[s * 4 + 3]),
    ]

  gdesc = {0: issue_gathers(0, 0)}
  odesc = {}
  for c in range(NCHUNK):
    s = c % 2
    b0 = w0 + c * NB
    if c + 1 < NCHUNK:
      if c >= 1:
        for d in odesc[c - 1]:
          d.wait()
      gdesc[c + 1] = issue_gathers(c + 1, (c + 1) % 2)

    g = gdesc.pop(c)
    out = []
    g[0].wait()
    out.append(pltpu.async_copy(idrows_v.at[s],
                                emb_id_out.at[pl.ds(b0 * NUM_ID, IDN)],
                                os_[s * 4 + 0]))
    g[2].wait()
    out.append(pltpu.async_copy(wide_id_v.at[s],
                                wide_id_out.at[pl.ds(b0 * NUM_ID, IDN)],
                                os_[s * 4 + 1]))

    g[1].wait()

    def ksum(k, carry):
      rows = tagrows_v.at[s]
      sums = tagsum_v.at[s]
      for t in range(NUM_TAG):
        base = k * TPB + t * HIST
        acc0 = jnp.zeros((16,), jnp.float32)
        acc1 = jnp.zeros((16,), jnp.float32)
        for j in range(HIST):
          acc0 = acc0 + rows[base + j, pl.ds(0, 16)]
          acc1 = acc1 + rows[base + j, pl.ds(16, 16)]
        sums[k * NUM_TAG + t, pl.ds(0, 16)] = acc0
        sums[k * NUM_TAG + t, pl.ds(16, 16)] = acc1
      return carry

    lax.fori_loop(0, NB, ksum, 0, unroll=False)
    out.append(pltpu.async_copy(tagsum_v.at[s],
                                emb_tag_out.at[pl.ds(b0 * NUM_TAG,
                                                     NB * NUM_TAG)],
                                os_[s * 4 + 2]))

    g[3].wait()
    for t in range(NUM_TAG):
      acc = jnp.zeros((16,), jnp.float32)
      for j in range(HIST):
        acc = acc + wide_tag_v[s, pl.ds(t * HIST * NB + j * NB, 16)]
      widesum_v[s, pl.ds(t * NB, 16)] = acc
      out.append(pltpu.async_copy(widesum_v.at[s, pl.ds(t * NB, NB)],
                                  wide_tag_out.at[pl.ds(t * B + b0, NB)],
                                  os_[s * 4 + 3]))
    odesc[c] = out

  for d in odesc[NCHUNK - 2]:
    d.wait()
  for d in odesc[NCHUNK - 1]:
    d.wait()


@jax.jit
def _run(id_idx2, tag_idx2, tagt_idx2, emb_table, wide_weight):
  mesh = plsc.VectorSubcoreMesh(core_axis_name="c", subcore_axis_name="s",
                                num_cores=NC, num_subcores=NS)
  out_type = [
      jax.ShapeDtypeStruct((B * NUM_ID, EMB), jnp.float32),
      jax.ShapeDtypeStruct((B * NUM_TAG, EMB), jnp.float32),
      jax.ShapeDtypeStruct((B * NUM_ID,), jnp.float32),
      jax.ShapeDtypeStruct((NUM_TAG * B,), jnp.float32),
  ]
  scratch_types = [
      pltpu.VMEM((BPW * NUM_ID,), jnp.int32),
      pltpu.VMEM((BPW * TPB,), jnp.int32),
      pltpu.VMEM((BPW * TPB,), jnp.int32),
      pltpu.VMEM((2, IDN, EMB), jnp.float32),
      pltpu.VMEM((2, TAGN, EMB), jnp.float32),
      pltpu.VMEM((2, IDN), jnp.float32),
      pltpu.VMEM((2, TAGN), jnp.float32),
      pltpu.VMEM((2, NB * NUM_TAG, EMB), jnp.float32),
      pltpu.VMEM((2, NUM_TAG * NB), jnp.float32),
  ] + [pltpu.SemaphoreType.DMA] * 16
  run = pl.kernel(_sc_body, out_type=out_type, mesh=mesh,
                  scratch_types=scratch_types,
                  compiler_params=pltpu.CompilerParams(
                      use_tc_tiling_on_sc=False))
  return run(id_idx2, tag_idx2, tagt_idx2, emb_table, wide_weight)


def kernel(feat_0, feat_1, feat_2, feat_3, feat_4, feat_5, feat_6, feat_7,
           feat_8, feat_9, feat_10, feat_11, feat_12, feat_13, feat_14,
           feat_15, feat_16, feat_17, feat_18, feat_19, feat_20, feat_21,
           feat_22, feat_23, tag_0, tag_1, emb_table, wide_weight):
  feats = [feat_0, feat_1, feat_2, feat_3, feat_4, feat_5, feat_6, feat_7,
           feat_8, feat_9, feat_10, feat_11, feat_12, feat_13, feat_14,
           feat_15, feat_16, feat_17, feat_18, feat_19, feat_20, feat_21,
           feat_22, feat_23]
  id_idx = jnp.stack(feats, axis=1).reshape(B * NUM_ID)
  tags = jnp.stack([tag_0, tag_1], axis=1)
  tag_idx = tags.reshape(B * TPB)
  tagt_idx = tags.reshape(B // NB, NB, NUM_TAG, HIST).transpose(
      0, 2, 3, 1).reshape(B * TPB)

  emb_id, emb_tag, wide_id, wide_tag = _run(id_idx, tag_idx, tagt_idx,
                                            emb_table, wide_weight)
  emb = jnp.concatenate([emb_id.reshape(B, NUM_ID, EMB),
                         emb_tag.reshape(B, NUM_TAG, EMB)], axis=1)
  wide = jnp.concatenate([wide_id.reshape(B, NUM_ID),
                          wide_tag.reshape(NUM_TAG, B).T], axis=1)
  return (wide, emb)
